# baseline (speedup 1.0000x reference)
import jax
import jax.numpy as jnp
from jax.experimental import pallas as pl
from jax.experimental.pallas import tpu as pltpu

B, S, H = 1, 2048, 1024
I, R, E, K = 1024, 256, 8, 2
TM = 2048
IC = 128
NC = I // IC


def _moe_body(x_ref, wg_ref, ga_ref, gb_ref, ua_ref, ub_ref, da_ref, db_ref,
              out_ref, logits_ref, xb_ref, w1_ref, w2_ref, i1_ref, i2_ref):
    e = pl.program_id(0)
    bf = jnp.bfloat16

    @pl.when(e == 0)
    def _():
        x = x_ref[...]
        logits = jax.lax.dot_general(
            x, wg_ref[...], (((1,), (1,)), ((), ())),
            preferred_element_type=jnp.float32)
        logits_ref[...] = logits
        p = jax.nn.softmax(logits, axis=-1)
        lane = jax.lax.broadcasted_iota(jnp.int32, (TM, E), 1)
        v1 = jnp.max(p, axis=-1, keepdims=True)
        idx1 = jnp.min(jnp.where(p == v1, lane, E), axis=-1, keepdims=True)
        p2 = jnp.where(lane == idx1, -1.0, p)
        v2 = jnp.max(p2, axis=-1, keepdims=True)
        idx2 = jnp.min(jnp.where(p2 == v2, lane, E), axis=-1, keepdims=True)
        denom = v1 + v2
        w1_ref[...] = v1 / denom
        w2_ref[...] = v2 / denom
        i1_ref[...] = idx1
        i2_ref[...] = idx2
        xb_ref[...] = x.astype(bf)
        out_ref[...] = jnp.zeros_like(out_ref)

    wcol = (jnp.where(i1_ref[...] == e, w1_ref[...], 0.0)
            + jnp.where(i2_ref[...] == e, w2_ref[...], 0.0))

    xb = xb_ref[...]
    aw = jnp.concatenate([ga_ref[0], ua_ref[0]], axis=0).astype(bf)

    dims = (((1,), (1,)), ((), ()))
    a_gu = jax.lax.dot_general(xb, aw, dims,
                               preferred_element_type=jnp.float32).astype(bf)
    a_g = a_gu[:, :R]
    a_u = a_gu[:, R:]

    b_d = jnp.zeros((TM, R), dtype=jnp.float32)
    for c in range(NC):
        gb_c = gb_ref[0, c * IC:(c + 1) * IC, :].astype(bf)
        ub_c = ub_ref[0, c * IC:(c + 1) * IC, :].astype(bf)
        da_c = da_ref[0, :, c * IC:(c + 1) * IC].astype(bf)
        g_c = jax.lax.dot_general(a_g, gb_c, dims,
                                  preferred_element_type=jnp.float32)
        u_c = jax.lax.dot_general(a_u, ub_c, dims,
                                  preferred_element_type=jnp.float32)
        h_c = ((g_c * jax.nn.sigmoid(g_c)) * u_c).astype(bf)
        b_d = b_d + jax.lax.dot_general(h_c, da_c, dims,
                                        preferred_element_type=jnp.float32)

    db = db_ref[0].astype(bf)
    dwn = jax.lax.dot_general(b_d.astype(bf), db, dims,
                              preferred_element_type=jnp.float32)

    out_ref[...] += wcol * dwn


def kernel(hidden_states, Wg, gate_A, gate_B, up_A, up_B, down_A, down_B):
    x = hidden_states.reshape(-1, H)

    out, logits = pl.pallas_call(
        _moe_body,
        grid=(E,),
        in_specs=[
            pl.BlockSpec((TM, H), lambda e: (0, 0)),
            pl.BlockSpec((E, H), lambda e: (0, 0)),
            pl.BlockSpec((1, R, H), lambda e: (e, 0, 0)),
            pl.BlockSpec((1, I, R), lambda e: (e, 0, 0)),
            pl.BlockSpec((1, R, H), lambda e: (e, 0, 0)),
            pl.BlockSpec((1, I, R), lambda e: (e, 0, 0)),
            pl.BlockSpec((1, R, I), lambda e: (e, 0, 0)),
            pl.BlockSpec((1, H, R), lambda e: (e, 0, 0)),
        ],
        out_specs=[
            pl.BlockSpec((TM, H), lambda e: (0, 0)),
            pl.BlockSpec((TM, E), lambda e: (0, 0)),
        ],
        out_shape=[
            jax.ShapeDtypeStruct((S, H), jnp.float32),
            jax.ShapeDtypeStruct((S, E), jnp.float32),
        ],
        scratch_shapes=[
            pltpu.VMEM((TM, H), jnp.bfloat16),
            pltpu.VMEM((TM, 1), jnp.float32),
            pltpu.VMEM((TM, 1), jnp.float32),
            pltpu.VMEM((TM, 1), jnp.int32),
            pltpu.VMEM((TM, 1), jnp.int32),
        ],
        compiler_params=pltpu.CompilerParams(
            dimension_semantics=("arbitrary",),
            vmem_limit_bytes=112 * 1024 * 1024,
        ),
    )(x, Wg, gate_A, gate_B, up_A, up_B, down_A, down_B)

    return out.reshape(B, S, H), logits


# speedup vs baseline: 1.3323x; 1.3323x over previous
import jax
import jax.numpy as jnp
from jax.experimental import pallas as pl
from jax.experimental.pallas import tpu as pltpu

B, S, H = 1, 2048, 1024
I, R, E, K = 1024, 256, 8, 2
TM = 2048
IC = 256
NC = I // IC


def _moe_body(x_ref, wg_ref, ga_ref, gb_ref, ua_ref, ub_ref, da_ref, db_ref,
              out_ref, logits_ref, xb_ref, w1_ref, w2_ref, i1_ref, i2_ref):
    e = pl.program_id(0)
    bf = jnp.bfloat16

    @pl.when(e == 0)
    def _():
        x = x_ref[...]
        logits = jax.lax.dot_general(
            x, wg_ref[...], (((1,), (1,)), ((), ())),
            preferred_element_type=jnp.float32)
        logits_ref[...] = logits
        p = jax.nn.softmax(logits, axis=-1)
        lane = jax.lax.broadcasted_iota(jnp.int32, (TM, E), 1)
        v1 = jnp.max(p, axis=-1, keepdims=True)
        idx1 = jnp.min(jnp.where(p == v1, lane, E), axis=-1, keepdims=True)
        p2 = jnp.where(lane == idx1, -1.0, p)
        v2 = jnp.max(p2, axis=-1, keepdims=True)
        idx2 = jnp.min(jnp.where(p2 == v2, lane, E), axis=-1, keepdims=True)
        denom = v1 + v2
        w1_ref[...] = v1 / denom
        w2_ref[...] = v2 / denom
        i1_ref[...] = idx1
        i2_ref[...] = idx2
        xb_ref[...] = x.astype(bf)
        out_ref[...] = jnp.zeros_like(out_ref)

    wcol = (jnp.where(i1_ref[...] == e, w1_ref[...], 0.0)
            + jnp.where(i2_ref[...] == e, w2_ref[...], 0.0))

    xb = xb_ref[...]
    aw = jnp.concatenate([ga_ref[0], ua_ref[0]], axis=0).astype(bf)

    dims = (((1,), (1,)), ((), ()))
    a_gu = jax.lax.dot_general(xb, aw, dims,
                               preferred_element_type=jnp.float32).astype(bf)
    a_g = a_gu[:, :R]
    a_u = a_gu[:, R:]

    b_d = jnp.zeros((TM, R), dtype=jnp.float32)
    for c in range(NC):
        gb_c = gb_ref[0, c * IC:(c + 1) * IC, :].astype(bf)
        ub_c = ub_ref[0, c * IC:(c + 1) * IC, :].astype(bf)
        da_c = da_ref[0, :, c * IC:(c + 1) * IC].astype(bf)
        g_c = jax.lax.dot_general(a_g, gb_c, dims,
                                  preferred_element_type=jnp.float32)
        u_c = jax.lax.dot_general(a_u, ub_c, dims,
                                  preferred_element_type=jnp.float32)
        h_c = ((g_c * jax.nn.sigmoid(g_c)) * u_c).astype(bf)
        b_d = b_d + jax.lax.dot_general(h_c, da_c, dims,
                                        preferred_element_type=jnp.float32)

    db = db_ref[0].astype(bf)
    dwn = jax.lax.dot_general(b_d.astype(bf), db, dims,
                              preferred_element_type=jnp.float32)

    out_ref[...] += wcol * dwn


def kernel(hidden_states, Wg, gate_A, gate_B, up_A, up_B, down_A, down_B):
    x = hidden_states.reshape(-1, H)

    out, logits = pl.pallas_call(
        _moe_body,
        grid=(E,),
        in_specs=[
            pl.BlockSpec((TM, H), lambda e: (0, 0)),
            pl.BlockSpec((E, H), lambda e: (0, 0)),
            pl.BlockSpec((1, R, H), lambda e: (e, 0, 0)),
            pl.BlockSpec((1, I, R), lambda e: (e, 0, 0)),
            pl.BlockSpec((1, R, H), lambda e: (e, 0, 0)),
            pl.BlockSpec((1, I, R), lambda e: (e, 0, 0)),
            pl.BlockSpec((1, R, I), lambda e: (e, 0, 0)),
            pl.BlockSpec((1, H, R), lambda e: (e, 0, 0)),
        ],
        out_specs=[
            pl.BlockSpec((TM, H), lambda e: (0, 0)),
            pl.BlockSpec((TM, E), lambda e: (0, 0)),
        ],
        out_shape=[
            jax.ShapeDtypeStruct((S, H), jnp.float32),
            jax.ShapeDtypeStruct((S, E), jnp.float32),
        ],
        scratch_shapes=[
            pltpu.VMEM((TM, H), jnp.bfloat16),
            pltpu.VMEM((TM, 1), jnp.float32),
            pltpu.VMEM((TM, 1), jnp.float32),
            pltpu.VMEM((TM, 1), jnp.int32),
            pltpu.VMEM((TM, 1), jnp.int32),
        ],
        compiler_params=pltpu.CompilerParams(
            dimension_semantics=("arbitrary",),
            vmem_limit_bytes=112 * 1024 * 1024,
        ),
    )(x, Wg, gate_A, gate_B, up_A, up_B, down_A, down_B)

    return out.reshape(B, S, H), logits


# speedup vs baseline: 1.3460x; 1.0103x over previous
import jax
import jax.numpy as jnp
from jax.experimental import pallas as pl
from jax.experimental.pallas import tpu as pltpu

B, S, H = 1, 2048, 1024
I, R, E, K = 1024, 256, 8, 2
TM = 2048
IC = 256
NC = I // IC


def _moe_body(x_ref, wg_ref, ga_ref, gb_ref, ua_ref, ub_ref, da_ref, db_ref,
              out_ref, logits_ref, xb_ref, w1_ref, w2_ref, i1_ref, i2_ref):
    e = pl.program_id(0)
    bf = jnp.bfloat16

    @pl.when(e == 0)
    def _():
        x = x_ref[...]
        logits = jax.lax.dot_general(
            x, wg_ref[...], (((1,), (1,)), ((), ())),
            preferred_element_type=jnp.float32)
        logits_ref[...] = logits
        p = jax.nn.softmax(logits, axis=-1)
        lane = jax.lax.broadcasted_iota(jnp.int32, (TM, E), 1)
        v1 = jnp.max(p, axis=-1, keepdims=True)
        idx1 = jnp.min(jnp.where(p == v1, lane, E), axis=-1, keepdims=True)
        p2 = jnp.where(lane == idx1, -1.0, p)
        v2 = jnp.max(p2, axis=-1, keepdims=True)
        idx2 = jnp.min(jnp.where(p2 == v2, lane, E), axis=-1, keepdims=True)
        denom = v1 + v2
        w1_ref[...] = v1 / denom
        w2_ref[...] = v2 / denom
        i1_ref[...] = idx1
        i2_ref[...] = idx2
        out_ref[...] = jnp.zeros_like(out_ref)

    wcol = (jnp.where(i1_ref[...] == e, w1_ref[...], 0.0)
            + jnp.where(i2_ref[...] == e, w2_ref[...], 0.0))

    xb = x_ref[...]
    aw = jnp.concatenate([ga_ref[0], ua_ref[0]], axis=0)

    dims = (((1,), (1,)), ((), ()))
    a_gu = jax.lax.dot_general(xb, aw, dims,
                               preferred_element_type=jnp.float32)
    a_g = a_gu[:, :R]
    a_u = a_gu[:, R:]

    b_d = jnp.zeros((TM, R), dtype=jnp.float32)
    for c in range(NC):
        gb_c = gb_ref[0, c * IC:(c + 1) * IC, :]
        ub_c = ub_ref[0, c * IC:(c + 1) * IC, :]
        da_c = da_ref[0, :, c * IC:(c + 1) * IC]
        g_c = jax.lax.dot_general(a_g, gb_c, dims,
                                  preferred_element_type=jnp.float32)
        u_c = jax.lax.dot_general(a_u, ub_c, dims,
                                  preferred_element_type=jnp.float32)
        h_c = (g_c * jax.nn.sigmoid(g_c)) * u_c
        b_d = b_d + jax.lax.dot_general(h_c, da_c, dims,
                                        preferred_element_type=jnp.float32)

    db = db_ref[0]
    dwn = jax.lax.dot_general(b_d, db, dims,
                              preferred_element_type=jnp.float32)

    out_ref[...] += wcol * dwn


def kernel(hidden_states, Wg, gate_A, gate_B, up_A, up_B, down_A, down_B):
    x = hidden_states.reshape(-1, H)

    out, logits = pl.pallas_call(
        _moe_body,
        grid=(E,),
        in_specs=[
            pl.BlockSpec((TM, H), lambda e: (0, 0)),
            pl.BlockSpec((E, H), lambda e: (0, 0)),
            pl.BlockSpec((1, R, H), lambda e: (e, 0, 0)),
            pl.BlockSpec((1, I, R), lambda e: (e, 0, 0)),
            pl.BlockSpec((1, R, H), lambda e: (e, 0, 0)),
            pl.BlockSpec((1, I, R), lambda e: (e, 0, 0)),
            pl.BlockSpec((1, R, I), lambda e: (e, 0, 0)),
            pl.BlockSpec((1, H, R), lambda e: (e, 0, 0)),
        ],
        out_specs=[
            pl.BlockSpec((TM, H), lambda e: (0, 0)),
            pl.BlockSpec((TM, E), lambda e: (0, 0)),
        ],
        out_shape=[
            jax.ShapeDtypeStruct((S, H), jnp.float32),
            jax.ShapeDtypeStruct((S, E), jnp.float32),
        ],
        scratch_shapes=[
            pltpu.VMEM((TM, H), jnp.bfloat16),
            pltpu.VMEM((TM, 1), jnp.float32),
            pltpu.VMEM((TM, 1), jnp.float32),
            pltpu.VMEM((TM, 1), jnp.int32),
            pltpu.VMEM((TM, 1), jnp.int32),
        ],
        compiler_params=pltpu.CompilerParams(
            dimension_semantics=("arbitrary",),
            vmem_limit_bytes=112 * 1024 * 1024,
        ),
    )(x, Wg, gate_A, gate_B, up_A, up_B, down_A, down_B)

    return out.reshape(B, S, H), logits
